# Initial kernel scaffold; baseline (speedup 1.0000x reference)
import jax
import jax.numpy as jnp
from jax.experimental import pallas as pl


def kernel(tree, idxes, w1, b1, w2, b2, w3, b3):
    raise NotImplementedError("write your pallas kernel here")


# speedup vs baseline: 2212.9254x; 2212.9254x over previous
import jax
import jax.numpy as jnp
from jax.experimental import pallas as pl

_BB = 8


def _tcnn_block_kernel(tree_ref, idxp_ref, w1_ref, w2_ref, w3_ref,
                       bm1_ref, bm2_ref, bm3_ref, out_ref):
    N = 128
    rows = jax.lax.broadcasted_iota(jnp.int32, (N, N), 0)

    def tln(x):
        n = x.shape[0] * x.shape[1]
        mean = jnp.sum(x) / n
        xc = x - mean
        var = jnp.sum(xc * xc) / (n - 1)
        return xc / (jnp.sqrt(var) + 1e-5)

    def leaky(x):
        return jnp.where(x >= 0, x, 0.01 * x)

    for s in range(_BB):
        h = tree_ref[s]
        sel = []
        for k in range(3):
            idxrow = idxp_ref[s, k:k + 1, :]
            sel.append((rows == idxrow).astype(jnp.float32))

        def layer(h, w_ref, bm_ref, cout):
            y = jnp.dot(w_ref[...], h, preferred_element_type=jnp.float32)
            acc = bm_ref[...]
            for k in range(3):
                acc = acc + jnp.dot(y[k * cout:(k + 1) * cout], sel[k],
                                    preferred_element_type=jnp.float32)
            return acc

        h1 = leaky(tln(layer(h, w1_ref, bm1_ref, 256)))
        h2 = leaky(tln(layer(h1, w2_ref, bm2_ref, 128)))
        h3 = tln(layer(h2, w3_ref, bm3_ref, 64))
        pooled = jnp.max(h3, axis=1, keepdims=True)
        out_ref[s:s + 1, :] = jnp.sum(pooled, axis=0, keepdims=True)


def kernel(tree, idxes, w1, b1, w2, b2, w3, b3):
    B, cin, N = tree.shape
    idx = idxes[..., 0].astype(jnp.int32)
    idx_t = jnp.transpose(idx.reshape(B, N - 1, 3), (0, 2, 1))
    idxp = jnp.concatenate(
        [jnp.full((B, 3, 1), -1, jnp.int32), idx_t], axis=2)

    w1r = jnp.transpose(w1, (2, 0, 1)).reshape(3 * 256, cin)
    w2r = jnp.transpose(w2, (2, 0, 1)).reshape(3 * 128, 256)
    w3r = jnp.transpose(w3, (2, 0, 1)).reshape(3 * 64, 128)
    colmask = (jnp.arange(N) > 0).astype(jnp.float32)[None, :]
    bm1 = b1[:, None] * colmask
    bm2 = b2[:, None] * colmask
    bm3 = b3[:, None] * colmask

    grid = (B // _BB,)
    out = pl.pallas_call(
        _tcnn_block_kernel,
        grid=grid,
        in_specs=[
            pl.BlockSpec((_BB, cin, N), lambda i: (i, 0, 0)),
            pl.BlockSpec((_BB, 3, N), lambda i: (i, 0, 0)),
            pl.BlockSpec(w1r.shape, lambda i: (0, 0)),
            pl.BlockSpec(w2r.shape, lambda i: (0, 0)),
            pl.BlockSpec(w3r.shape, lambda i: (0, 0)),
            pl.BlockSpec(bm1.shape, lambda i: (0, 0)),
            pl.BlockSpec(bm2.shape, lambda i: (0, 0)),
            pl.BlockSpec(bm3.shape, lambda i: (0, 0)),
        ],
        out_specs=pl.BlockSpec((_BB, 1), lambda i: (i, 0)),
        out_shape=jax.ShapeDtypeStruct((B, 1), jnp.float32),
    )(tree, idxp, w1r, w2r, w3r, bm1, bm2, bm3)
    return out


# speedup vs baseline: 4206.9254x; 1.9011x over previous
import jax
import jax.numpy as jnp
from jax.experimental import pallas as pl

_BB = 8
_N = 128


def _tln(x):
    n = x.shape[0] * x.shape[1]
    mean = jnp.sum(x) / n
    xc = x - mean
    var = jnp.sum(xc * xc) / (n - 1)
    return xc / (jnp.sqrt(var) + 1e-5)


def _leaky(x):
    return jnp.where(x >= 0, x, 0.01 * x)


def _tcnn_block_kernel(tree_ref, idxp_ref, w1_ref, w2_ref, w3_ref,
                       bm1_ref, bm2_ref, bm3_ref, out_ref):
    N = _N
    f32 = jnp.float32
    rows = jax.lax.broadcasted_iota(jnp.int32, (N, N), 0)
    sel = [[(rows == idxp_ref[s, k:k + 1, :]).astype(f32) for k in range(3)]
           for s in range(_BB)]

    acc = bm1_ref[...]
    for k in range(3):
        ek = jnp.concatenate(
            [jnp.dot(tree_ref[s], sel[s][k], preferred_element_type=f32)
             for s in range(_BB)], axis=1)
        acc = acc + jnp.dot(w1_ref[k * 256:(k + 1) * 256], ek,
                            preferred_element_type=f32)
    h1 = _leaky(jnp.concatenate(
        [_tln(acc[:, s * N:(s + 1) * N]) for s in range(_BB)], axis=1))

    y2 = jnp.dot(w2_ref[...], h1, preferred_element_type=f32)
    h2s = []
    for s in range(_BB):
        a = bm2_ref[...]
        for k in range(3):
            a = a + jnp.dot(y2[k * 128:(k + 1) * 128, s * N:(s + 1) * N],
                            sel[s][k], preferred_element_type=f32)
        h2s.append(_tln(a))
    h2 = _leaky(jnp.concatenate(h2s, axis=1))

    y3 = jnp.dot(w3_ref[...], h2, preferred_element_type=f32)
    for s in range(_BB):
        a = bm3_ref[...]
        for k in range(3):
            a = a + jnp.dot(y3[k * 64:(k + 1) * 64, s * N:(s + 1) * N],
                            sel[s][k], preferred_element_type=f32)
        a = _tln(a)
        pooled = jnp.max(a, axis=1, keepdims=True)
        out_ref[s:s + 1, :] = jnp.sum(pooled, axis=0, keepdims=True)


def kernel(tree, idxes, w1, b1, w2, b2, w3, b3):
    B, cin, N = tree.shape
    idx = idxes[..., 0].astype(jnp.int32)
    idx_t = jnp.transpose(idx.reshape(B, N - 1, 3), (0, 2, 1))
    idxp = jnp.concatenate(
        [jnp.full((B, 3, 1), -1, jnp.int32), idx_t], axis=2)

    w1r = jnp.transpose(w1, (2, 0, 1)).reshape(3 * 256, cin)
    w2r = jnp.transpose(w2, (2, 0, 1)).reshape(3 * 128, 256)
    w3r = jnp.transpose(w3, (2, 0, 1)).reshape(3 * 64, 128)
    colmask = (jnp.arange(N) > 0).astype(jnp.float32)[None, :]
    bm1 = jnp.tile(b1[:, None] * colmask, (1, _BB))
    bm2 = b2[:, None] * colmask
    bm3 = b3[:, None] * colmask

    grid = (B // _BB,)
    out = pl.pallas_call(
        _tcnn_block_kernel,
        grid=grid,
        in_specs=[
            pl.BlockSpec((_BB, cin, N), lambda i: (i, 0, 0)),
            pl.BlockSpec((_BB, 3, N), lambda i: (i, 0, 0)),
            pl.BlockSpec(w1r.shape, lambda i: (0, 0)),
            pl.BlockSpec(w2r.shape, lambda i: (0, 0)),
            pl.BlockSpec(w3r.shape, lambda i: (0, 0)),
            pl.BlockSpec(bm1.shape, lambda i: (0, 0)),
            pl.BlockSpec(bm2.shape, lambda i: (0, 0)),
            pl.BlockSpec(bm3.shape, lambda i: (0, 0)),
        ],
        out_specs=pl.BlockSpec((_BB, 1), lambda i: (i, 0)),
        out_shape=jax.ShapeDtypeStruct((B, 1), jnp.float32),
    )(tree, idxp, w1r, w2r, w3r, bm1, bm2, bm3)
    return out


# speedup vs baseline: 5324.0293x; 1.2655x over previous
import jax
import jax.numpy as jnp
from jax.experimental import pallas as pl

_BB = 16
_GS = 4
_N = 128


def _tln(x):
    n = x.shape[0] * x.shape[1]
    s1 = jnp.sum(x)
    s2 = jnp.sum(x * x)
    mean = s1 / n
    var = jnp.maximum((s2 - s1 * mean), 0.0) / (n - 1)
    rinv = 1.0 / (jnp.sqrt(var) + 1e-5)
    return x * rinv - mean * rinv


def _leaky(x):
    return jnp.where(x >= 0, x, 0.01 * x)


def _tcnn_block_kernel(tree_ref, idxp_ref, w1_ref, w2_ref, w3_ref,
                       bm1_ref, bm2_ref, bm3_ref, out_ref):
    N = _N
    f32 = jnp.float32
    rows = jax.lax.broadcasted_iota(jnp.int32, (N, N), 0)
    sel = [[(rows == idxp_ref[s, k:k + 1, :]).astype(f32) for k in range(3)]
           for s in range(_BB)]

    for g in range(_BB // _GS):
        smp = range(g * _GS, (g + 1) * _GS)

        acc = bm1_ref[...]
        for k in range(3):
            ek = jnp.concatenate(
                [jnp.dot(tree_ref[s], sel[s][k], preferred_element_type=f32)
                 for s in smp], axis=1)
            acc = acc + jnp.dot(w1_ref[k * 256:(k + 1) * 256], ek,
                                preferred_element_type=f32)
        h1 = _leaky(jnp.concatenate(
            [_tln(acc[:, j * N:(j + 1) * N]) for j in range(_GS)], axis=1))

        y2 = jnp.dot(w2_ref[...], h1, preferred_element_type=f32)
        h2s = []
        for j, s in enumerate(smp):
            a = bm2_ref[...]
            for k in range(3):
                a = a + jnp.dot(y2[k * 128:(k + 1) * 128, j * N:(j + 1) * N],
                                sel[s][k], preferred_element_type=f32)
            h2s.append(_tln(a))
        h2 = _leaky(jnp.concatenate(h2s, axis=1))

        y3 = jnp.dot(w3_ref[...], h2, preferred_element_type=f32)
        for j, s in enumerate(smp):
            a = bm3_ref[...]
            for k in range(3):
                a = a + jnp.dot(y3[k * 64:(k + 1) * 64, j * N:(j + 1) * N],
                                sel[s][k], preferred_element_type=f32)
            a = _tln(a)
            pooled = jnp.max(a, axis=1, keepdims=True)
            out_ref[s:s + 1, :] = jnp.sum(pooled, axis=0, keepdims=True)


def kernel(tree, idxes, w1, b1, w2, b2, w3, b3):
    B, cin, N = tree.shape
    idx = idxes[..., 0].astype(jnp.int32)
    idx_t = jnp.transpose(idx.reshape(B, N - 1, 3), (0, 2, 1))
    idxp = jnp.concatenate(
        [jnp.full((B, 3, 1), -1, jnp.int32), idx_t], axis=2)

    w1r = jnp.transpose(w1, (2, 0, 1)).reshape(3 * 256, cin)
    w2r = jnp.transpose(w2, (2, 0, 1)).reshape(3 * 128, 256)
    w3r = jnp.transpose(w3, (2, 0, 1)).reshape(3 * 64, 128)
    colmask = (jnp.arange(N) > 0).astype(jnp.float32)[None, :]
    bm1 = jnp.tile(b1[:, None] * colmask, (1, _GS))
    bm2 = b2[:, None] * colmask
    bm3 = b3[:, None] * colmask

    grid = (B // _BB,)
    out = pl.pallas_call(
        _tcnn_block_kernel,
        grid=grid,
        in_specs=[
            pl.BlockSpec((_BB, cin, N), lambda i: (i, 0, 0)),
            pl.BlockSpec((_BB, 3, N), lambda i: (i, 0, 0)),
            pl.BlockSpec(w1r.shape, lambda i: (0, 0)),
            pl.BlockSpec(w2r.shape, lambda i: (0, 0)),
            pl.BlockSpec(w3r.shape, lambda i: (0, 0)),
            pl.BlockSpec(bm1.shape, lambda i: (0, 0)),
            pl.BlockSpec(bm2.shape, lambda i: (0, 0)),
            pl.BlockSpec(bm3.shape, lambda i: (0, 0)),
        ],
        out_specs=pl.BlockSpec((_BB, 1), lambda i: (i, 0)),
        out_shape=jax.ShapeDtypeStruct((B, 1), jnp.float32),
    )(tree, idxp, w1r, w2r, w3r, bm1, bm2, bm3)
    return out


# speedup vs baseline: 10042.6492x; 1.8863x over previous
import jax
import jax.numpy as jnp
from jax.experimental import pallas as pl

_BB = 64
_GS = 16
_N = 128


def _stats(cs, cs2, j, n):
    s1 = jnp.sum(cs[:, j * _N:(j + 1) * _N])
    s2 = jnp.sum(cs2[:, j * _N:(j + 1) * _N])
    mean = s1 / n
    var = jnp.maximum(s2 - s1 * mean, 0.0) / (n - 1)
    rinv = 1.0 / (jnp.sqrt(var) + 1e-5)
    return rinv, -mean * rinv


def _norm_leaky_group(x, gs):
    n = x.shape[0] * _N
    cs = jnp.sum(x, axis=0, keepdims=True)
    cs2 = jnp.sum(x * x, axis=0, keepdims=True)
    scales, shifts = [], []
    for j in range(gs):
        rinv, shift = _stats(cs, cs2, j, n)
        scales.append(jnp.full((1, _N), rinv, jnp.float32))
        shifts.append(jnp.full((1, _N), shift, jnp.float32))
    sc = jnp.concatenate(scales, axis=1)
    sh = jnp.concatenate(shifts, axis=1)
    y = x * sc + sh
    return jnp.maximum(y, 0.01 * y)


def _tcnn_block_kernel(tree_ref, idxp_ref, w1_ref, w2_ref, w3_ref,
                       bm1_ref, bm2_ref, bm3_ref, out_ref):
    N = _N
    f32 = jnp.float32
    rows = jax.lax.broadcasted_iota(jnp.int32, (N, 3 * N), 0)
    ngroups = _BB // _GS
    groups = [range(g * _GS, (g + 1) * _GS) for g in range(ngroups)]
    selcat = [(rows == idxp_ref[s]).astype(f32) for s in range(_BB)]
    sel = [[selcat[s][:, k * N:(k + 1) * N] for k in range(3)]
           for s in range(_BB)]


    acc1 = []
    for smp in groups:
        acc = bm1_ref[...]
        for k in range(3):
            ek = jnp.concatenate(
                [jnp.dot(tree_ref[s], sel[s][k], preferred_element_type=f32)
                 for s in smp], axis=1)
            acc = acc + jnp.dot(w1_ref[k * 256:(k + 1) * 256], ek,
                                preferred_element_type=f32)
        acc1.append(acc)
    h1 = [_norm_leaky_group(a, _GS) for a in acc1]

    y2 = [jnp.dot(w2_ref[...], h, preferred_element_type=f32) for h in h1]
    a2 = []
    for g, smp in enumerate(groups):
        a2s = []
        for j, s in enumerate(smp):
            a = bm2_ref[...]
            for k in range(3):
                a = a + jnp.dot(
                    y2[g][k * 128:(k + 1) * 128, j * N:(j + 1) * N],
                    sel[s][k], preferred_element_type=f32)
            a2s.append(a)
        a2.append(jnp.concatenate(a2s, axis=1))
    h2 = [_norm_leaky_group(a, _GS) for a in a2]

    y3 = [jnp.dot(w3_ref[...], h, preferred_element_type=f32) for h in h2]
    a3 = []
    for g, smp in enumerate(groups):
        a3s = []
        for j, s in enumerate(smp):
            a = bm3_ref[...]
            for k in range(3):
                a = a + jnp.dot(
                    y3[g][k * 64:(k + 1) * 64, j * N:(j + 1) * N],
                    sel[s][k], preferred_element_type=f32)
            a3s.append(a)
        a3.append(jnp.concatenate(a3s, axis=1))
    for g, smp in enumerate(groups):
        cs = jnp.sum(a3[g], axis=0, keepdims=True)
        cs2 = jnp.sum(a3[g] * a3[g], axis=0, keepdims=True)
        for j, s in enumerate(smp):
            rinv, shift = _stats(cs, cs2, j, 64 * N)
            pooled = jnp.max(a3[g][:, j * N:(j + 1) * N],
                             axis=1, keepdims=True)
            m = jnp.sum(pooled, axis=0, keepdims=True)
            out_ref[s:s + 1, :] = m * rinv + 64.0 * shift


def kernel(tree, idxes, w1, b1, w2, b2, w3, b3):
    B, cin, N = tree.shape
    idx = idxes[..., 0].astype(jnp.int32)
    idx_t = jnp.transpose(idx.reshape(B, N - 1, 3), (0, 2, 1))
    idxp = jnp.concatenate(
        [jnp.full((B, 3, 1), -1, jnp.int32), idx_t], axis=2)
    idxp = idxp.reshape(B, 1, 3 * N)

    w1r = jnp.transpose(w1, (2, 0, 1)).reshape(3 * 256, cin)
    w2r = jnp.transpose(w2, (2, 0, 1)).reshape(3 * 128, 256)
    w3r = jnp.transpose(w3, (2, 0, 1)).reshape(3 * 64, 128)
    colmask = (jnp.arange(N) > 0).astype(jnp.float32)[None, :]
    bm1 = jnp.tile(b1[:, None] * colmask, (1, _GS))
    bm2 = b2[:, None] * colmask
    bm3 = b3[:, None] * colmask

    grid = (B // _BB,)
    out = pl.pallas_call(
        _tcnn_block_kernel,
        grid=grid,
        in_specs=[
            pl.BlockSpec((_BB, cin, N), lambda i: (i, 0, 0)),
            pl.BlockSpec((_BB, 1, 3 * N), lambda i: (i, 0, 0)),
            pl.BlockSpec(w1r.shape, lambda i: (0, 0)),
            pl.BlockSpec(w2r.shape, lambda i: (0, 0)),
            pl.BlockSpec(w3r.shape, lambda i: (0, 0)),
            pl.BlockSpec(bm1.shape, lambda i: (0, 0)),
            pl.BlockSpec(bm2.shape, lambda i: (0, 0)),
            pl.BlockSpec(bm3.shape, lambda i: (0, 0)),
        ],
        out_specs=pl.BlockSpec((_BB, 1), lambda i: (i, 0)),
        out_shape=jax.ShapeDtypeStruct((B, 1), jnp.float32),
    )(tree, idxp, w1r, w2r, w3r, bm1, bm2, bm3)
    return out
